# Initial kernel scaffold; baseline (speedup 1.0000x reference)
import functools
import math

import jax
import jax.numpy as jnp
from jax import lax
from jax.experimental import pallas as pl
from jax.experimental.pallas import tpu as pltpu
from jax.experimental.pallas import tpu_sc as plsc

_V = 1024
_D = 256
_NF = 10
_W = _D + 2 * _NF
_B = 16384
_F = 3
_ND = _B * (_F - 1)
_NW = 32
_PER_W = _ND // _NW
_ROWS_W = _B // _NW
_C = 128
_NCHUNK = _PER_W // _C


def _table_body(emb_ref, out_ref):
    d = lax.broadcasted_iota(jnp.float32, (_V, _NF), 0)
    k = lax.broadcasted_iota(jnp.int32, (_V, _NF), 1)
    coef = (jnp.float32(math.pi) / jnp.float32(_V)) * (
        lax.shift_left(jnp.int32(1), k).astype(jnp.float32))
    raw = coef * d
    out_ref[...] = jnp.concatenate(
        [emb_ref[...], jnp.sin(raw), jnp.cos(raw)], axis=1)


def _build_table(embed_table):
    return pl.pallas_call(
        _table_body,
        out_shape=jax.ShapeDtypeStruct((_V, _W), jnp.float32),
    )(embed_table)


def _sc_body(table, t_flat, out, t_v, idx_v, buf0, buf1, gsem0, gsem1,
             osem0, osem1):
    wid = lax.axis_index("s") * 2 + lax.axis_index("c")
    tbase = wid * (_ROWS_W * _F)
    pltpu.sync_copy(t_flat.at[pl.ds(tbase, _ROWS_W * _F)], t_v)

    lane = lax.iota(jnp.int32, 16)
    for u in range(_PER_W // 16):
        j = lane + (u * 16)
        b = j >> 1
        f = j & 1
        lo = b * _F + f
        diff = plsc.load_gather(t_v, [lo + 1]) - plsc.load_gather(t_v, [lo])
        idx_v[u // (_C // 16), pl.ds((u % (_C // 16)) * 16, 16)] = diff

    bufs = (buf0, buf1)
    gsems = (gsem0, gsem1)
    osems = (osem0, osem1)
    obase = wid * _PER_W
    gh = [None] * _NCHUNK
    oh = [None] * _NCHUNK
    gh[0] = pltpu.async_copy(table.at[idx_v.at[0]], bufs[0], gsems[0])
    for c in range(_NCHUNK):
        p = c & 1
        gh[c].wait()
        if c + 1 < _NCHUNK:
            if c >= 1:
                oh[c - 1].wait()
            gh[c + 1] = pltpu.async_copy(
                table.at[idx_v.at[c + 1]], bufs[1 - p], gsems[1 - p])
        oh[c] = pltpu.make_async_copy(
            bufs[p], out.at[pl.ds(obase + c * _C, _C)], osems[p])
        oh[c].start()
    oh[_NCHUNK - 2].wait()
    oh[_NCHUNK - 1].wait()


@functools.partial(
    pl.kernel,
    out_type=jax.ShapeDtypeStruct((_ND, _W), jnp.float32),
    mesh=plsc.VectorSubcoreMesh(core_axis_name="c", subcore_axis_name="s"),
    scratch_types=[
        pltpu.VMEM((_ROWS_W * _F,), jnp.int32),
        pltpu.VMEM((_NCHUNK, _C), jnp.int32),
        pltpu.VMEM((_C, _W), jnp.float32),
        pltpu.VMEM((_C, _W), jnp.float32),
        pltpu.SemaphoreType.DMA,
        pltpu.SemaphoreType.DMA,
        pltpu.SemaphoreType.DMA,
        pltpu.SemaphoreType.DMA,
    ],
)
def _sc_gather(table, t_flat, out, *rest):
    _sc_body(table, t_flat, out, *rest)


def kernel(t, embed_table):
    table = _build_table(embed_table)
    out = _sc_gather(table, t.reshape(-1))
    return out.reshape(_B, (_F - 1) * _W)


# speedup vs baseline: 1.2645x; 1.2645x over previous
import functools
import math

import jax
import jax.numpy as jnp
from jax import lax
from jax.experimental import pallas as pl
from jax.experimental.pallas import tpu as pltpu
from jax.experimental.pallas import tpu_sc as plsc

_V = 1024
_D = 256
_NF = 10
_W = _D + 2 * _NF
_WP = 288
_B = 16384
_F = 3
_ND = _B * (_F - 1)
_NW = 32
_PER_W = _ND // _NW
_ROWS_W = _B // _NW
_C = 128
_NCHUNK = _PER_W // _C


def _table_body(emb_ref, out_ref):
    d = lax.broadcasted_iota(jnp.int32, (_V, _NF), 0).astype(jnp.float32)
    k = lax.broadcasted_iota(jnp.int32, (_V, _NF), 1)
    coef = (jnp.float32(math.pi) / jnp.float32(_V)) * (
        lax.shift_left(jnp.int32(1), k).astype(jnp.float32))
    raw = coef * d
    out_ref[...] = jnp.concatenate(
        [emb_ref[...], jnp.sin(raw), jnp.cos(raw),
         jnp.zeros((_V, _WP - _W), jnp.float32)], axis=1)


def _build_table(embed_table):
    return pl.pallas_call(
        _table_body,
        out_shape=jax.ShapeDtypeStruct((_V, _WP), jnp.float32),
    )(embed_table)


def _sc_body(table, t_flat, out, t_v, idx_v, buf0, buf1, obuf, gsem0, gsem1):
    wid = lax.axis_index("s") * 2 + lax.axis_index("c")
    tbase = wid * (_ROWS_W * _F)
    pltpu.sync_copy(t_flat.at[pl.ds(tbase, _ROWS_W * _F)], t_v)

    lane = lax.iota(jnp.int32, 16)
    for u in range(_PER_W // 16):
        j = lane + (u * 16)
        b = j >> 1
        f = j & 1
        lo = b * _F + f
        diff = plsc.load_gather(t_v, [lo + 1]) - plsc.load_gather(t_v, [lo])
        idx_v[u // (_C // 16), pl.ds((u % (_C // 16)) * 16, 16)] = diff

    bufs = (buf0, buf1)
    gsems = (gsem0, gsem1)
    obase = wid * _PER_W

    def _compact(buf):
        def row_copy(r, _):
            for k in range(_D // 16):
                obuf[r, pl.ds(k * 16, 16)] = buf[r, pl.ds(k * 16, 16)]
            obuf[r, pl.ds(_D, 16)] = buf[r, pl.ds(_D, 16)]
            obuf[r, pl.ds(_W - 16, 16)] = buf[r, pl.ds(_W - 16, 16)]
            return 0
        lax.fori_loop(0, _C, row_copy, 0)

    gh = [None] * _NCHUNK
    gh[0] = pltpu.async_copy(table.at[idx_v.at[0]], bufs[0], gsems[0])
    for c in range(_NCHUNK):
        p = c & 1
        if c + 1 < _NCHUNK:
            gh[c + 1] = pltpu.async_copy(
                table.at[idx_v.at[c + 1]], bufs[1 - p], gsems[1 - p])
        gh[c].wait()
        _compact(bufs[p])
        pltpu.sync_copy(obuf, out.at[pl.ds(obase + c * _C, _C)])


@functools.partial(
    pl.kernel,
    out_type=jax.ShapeDtypeStruct((_ND, _W), jnp.float32),
    mesh=plsc.VectorSubcoreMesh(core_axis_name="c", subcore_axis_name="s"),
    compiler_params=pltpu.CompilerParams(
        needs_layout_passes=False, use_tc_tiling_on_sc=False),
    scratch_types=[
        pltpu.VMEM((_ROWS_W * _F,), jnp.int32),
        pltpu.VMEM((_NCHUNK, _C), jnp.int32),
        pltpu.VMEM((_C, _WP), jnp.float32),
        pltpu.VMEM((_C, _WP), jnp.float32),
        pltpu.VMEM((_C, _W), jnp.float32),
        pltpu.SemaphoreType.DMA,
        pltpu.SemaphoreType.DMA,
    ],
)
def _sc_gather(table, t_flat, out, *rest):
    _sc_body(table, t_flat, out, *rest)


def kernel(t, embed_table):
    table = _build_table(embed_table)
    out = _sc_gather(table, t.reshape(-1))
    return out.reshape(_B, (_F - 1) * _W)


# speedup vs baseline: 2.4980x; 1.9755x over previous
import functools
import math

import jax
import jax.numpy as jnp
from jax import lax
from jax.experimental import pallas as pl
from jax.experimental.pallas import tpu as pltpu
from jax.experimental.pallas import tpu_sc as plsc

_V = 1024
_D = 256
_NF = 10
_W = _D + 2 * _NF
_B = 16384
_F = 3
_NW = 32
_ROWS_W = _B // _NW
_R = 32
_NCHUNK = _ROWS_W // _R


def _fourier(shape_rows):
    d = lax.broadcasted_iota(jnp.int32, (shape_rows, 2 * _NF), 0).astype(
        jnp.float32)
    k = lax.broadcasted_iota(jnp.int32, (shape_rows, 2 * _NF), 1)
    kk = k % _NF
    coef = (jnp.float32(math.pi) / jnp.float32(_V)) * (
        lax.shift_left(jnp.int32(1), kk).astype(jnp.float32))
    raw = coef * d
    return jnp.where(k < _NF, jnp.sin(raw), jnp.cos(raw))


def _tables_body(emb_ref, tb_ref, tc_ref, ftab_ref):
    four = _fourier(_V)
    tb_ref[...] = jnp.concatenate(
        [jnp.zeros((_V, 2 * _NF), jnp.float32), emb_ref[:, :_D - 2 * _NF]],
        axis=1)
    tc_ref[...] = jnp.concatenate(
        [emb_ref[:, _D - 2 * _NF:], four,
         jnp.zeros((_V, 128 - 4 * _NF), jnp.float32)], axis=1)
    dr = lax.broadcasted_iota(jnp.int32, (_V // 4, 128), 0)
    c = lax.broadcasted_iota(jnp.int32, (_V // 4, 128), 1)
    d = (4 * dr + c // 32).astype(jnp.float32)
    k = c % 32
    kk = k % _NF
    coef = (jnp.float32(math.pi) / jnp.float32(_V)) * (
        lax.shift_left(jnp.int32(1), kk).astype(jnp.float32))
    raw = coef * d
    val = jnp.where(k < _NF, jnp.sin(raw),
                    jnp.where(k < 2 * _NF, jnp.cos(raw), 0.0))
    ftab_ref[...] = val


def _build_tables(embed_table):
    return pl.pallas_call(
        _tables_body,
        out_shape=(
            jax.ShapeDtypeStruct((_V, _D), jnp.float32),
            jax.ShapeDtypeStruct((_V, 128), jnp.float32),
            jax.ShapeDtypeStruct((_V // 4, 128), jnp.float32),
        ),
    )(embed_table)


def _sc_body(emb, tb, tc, ftab, t_flat, out, t_v, idx_e, idx_o, ftab_v,
             obuf0, obuf1, cbuf0, cbuf1, sa0, sa1, sb0, sb1, sc0, sc1):
    wid = lax.axis_index("s") * 2 + lax.axis_index("c")
    tbase = wid * (_ROWS_W * _F)
    pltpu.sync_copy(t_flat.at[pl.ds(tbase, _ROWS_W * _F)], t_v)
    pltpu.sync_copy(ftab, ftab_v)

    lane = lax.iota(jnp.int32, 16)
    for u in range(_ROWS_W // 16):
        b = lane + (u * 16)
        lo = b * _F
        t0 = plsc.load_gather(t_v, [lo])
        t1 = plsc.load_gather(t_v, [lo + 1])
        t2 = plsc.load_gather(t_v, [lo + 2])
        cc = u // (_R // 16)
        off = (u % (_R // 16)) * 16
        idx_e[cc, pl.ds(off, 16)] = t1 - t0
        idx_o[cc, pl.ds(off, 16)] = t2 - t1

    obufs = (obuf0, obuf1)
    cbufs = (cbuf0, cbuf1)
    sas = (sa0, sa1)
    sbs = (sb0, sb1)
    scs = (sc0, sc1)
    orow_base = wid * _ROWS_W

    def _fire(c):
        p = c & 1
        ga = pltpu.async_copy(
            emb.at[idx_e.at[c]], obufs[p].at[:, pl.ds(0, _D)], sas[p])
        gb = pltpu.async_copy(
            tb.at[idx_o.at[c]], obufs[p].at[:, pl.ds(_D, _D)], sbs[p])
        gc = pltpu.async_copy(tc.at[idx_o.at[c]], cbufs[p], scs[p])
        return (ga, gb, gc)

    lane = lax.iota(jnp.int32, 16)

    def _repair(c):
        p = c & 1
        obuf = obufs[p]
        cbuf = cbufs[p]
        for s in range(_R // 16):
            rows = lane + (s * 16)
            d0v = idx_e[c, pl.ds(s * 16, 16)]
            frv = d0v >> 2
            fcv = (d0v & 3) * 32

            def f_fix(k, _):
                vals = plsc.load_gather(ftab_v, [frv, fcv + k])
                plsc.store_scatter(obuf, [rows, (lane * 0) + (_D + k)], vals)
                return 0

            def t_fix(k, _):
                vals = plsc.load_gather(cbuf, [rows, (lane * 0) + k])
                plsc.store_scatter(
                    obuf, [rows, (lane * 0) + (2 * _D + k)], vals)
                return 0

            lax.fori_loop(0, 2 * _NF, f_fix, 0)
            lax.fori_loop(0, 4 * _NF, t_fix, 0)

    gh = [None] * _NCHUNK
    gh[0] = _fire(0)
    for c in range(_NCHUNK):
        p = c & 1
        if c + 1 < _NCHUNK:
            gh[c + 1] = _fire(c + 1)
        for h in gh[c]:
            h.wait()
        _repair(c)
        pltpu.sync_copy(obufs[p], out.at[pl.ds(orow_base + c * _R, _R)])


@functools.partial(
    pl.kernel,
    out_type=jax.ShapeDtypeStruct((_B, 2 * _W), jnp.float32),
    mesh=plsc.VectorSubcoreMesh(core_axis_name="c", subcore_axis_name="s"),
    compiler_params=pltpu.CompilerParams(needs_layout_passes=False),
    scratch_types=[
        pltpu.VMEM((_ROWS_W * _F,), jnp.int32),
        pltpu.VMEM((_NCHUNK, _R), jnp.int32),
        pltpu.VMEM((_NCHUNK, _R), jnp.int32),
        pltpu.VMEM((_V // 4, 128), jnp.float32),
        pltpu.VMEM((_R, 2 * _W), jnp.float32),
        pltpu.VMEM((_R, 2 * _W), jnp.float32),
        pltpu.VMEM((_R, 128), jnp.float32),
        pltpu.VMEM((_R, 128), jnp.float32),
        pltpu.SemaphoreType.DMA,
        pltpu.SemaphoreType.DMA,
        pltpu.SemaphoreType.DMA,
        pltpu.SemaphoreType.DMA,
        pltpu.SemaphoreType.DMA,
        pltpu.SemaphoreType.DMA,
    ],
)
def _sc_gather(emb, tb, tc, ftab, t_flat, out, *rest):
    _sc_body(emb, tb, tc, ftab, t_flat, out, *rest)


def kernel(t, embed_table):
    tb, tc, ftab = _build_tables(embed_table)
    return _sc_gather(embed_table, tb, tc, ftab, t.reshape(-1))
